# Initial kernel scaffold; baseline (speedup 1.0000x reference)
import jax
import jax.numpy as jnp
from jax.experimental import pallas as pl


def kernel(edge_attr, dst_ports, tcp_flags, edge_index, batch, dst_port_table, tcp_flags_table, W_e1, W_s1, W_self1, b1, W_e2, W_s2, W_self2, b2, Wc, bc):
    raise NotImplementedError("write your pallas kernel here")


# speedup vs baseline: 2.7609x; 2.7609x over previous
import functools

import jax
import jax.numpy as jnp
from jax import lax
from jax.experimental import pallas as pl
from jax.experimental.pallas import tpu as pltpu
from jax.experimental.pallas import tpu_sc as plsc

N = 100000
E = 1600000
G = 64
H = 32
NT = 16
CH = 128
NCH = E // CH
NPT = N // NT
N_FULL_NCH = N // CH
N_TAIL = N - N_FULL_NCH * CH
TAIL_TILE = N_FULL_NCH % NT

_mesh = plsc.VectorSubcoreMesh(core_axis_name="c", subcore_axis_name="s")


def _c16(v):
    return jnp.full((16,), v, jnp.int32)


_SC_PARAMS = pltpu.CompilerParams(needs_layout_passes=False,
                                  use_tc_tiling_on_sc=False)


def _zero_agg(msg, agg, s):
    def zv(i, carry):
        msg[i] = jnp.zeros((16,), jnp.float32)
        return carry

    lax.fori_loop(0, 125, zv, 0)

    def zc(i, carry):
        pltpu.sync_copy(msg.at[pl.ds(0, 125)],
                        agg.at[pl.ds(s * NPT + i * 125, 125)])
        return carry

    lax.fori_loop(0, 50, zc, 0)


def _make_edge_kernel(layer2):
    scratch = [
        pltpu.VMEM_SHARED((N, 16), jnp.float32),
        pltpu.VMEM((CH,), jnp.int32),
        pltpu.VMEM((CH,), jnp.int32),
        pltpu.VMEM((1, CH), jnp.int32),
        pltpu.VMEM((CH, 4), jnp.float32),
        pltpu.VMEM((CH, 16), jnp.float32),
        pltpu.VMEM((CH, 16), jnp.float32),
        pltpu.VMEM((256, 16), jnp.float32),
        pltpu.VMEM((5, 16), jnp.float32),
        pltpu.SemaphoreType.DMA,
    ]
    if layer2:
        scratch = scratch + [
            pltpu.VMEM((CH,), jnp.int32),
            pltpu.VMEM((CH, 16), jnp.float32),
        ]

    @functools.partial(
        pl.kernel,
        mesh=_mesh,
        out_type=jax.ShapeDtypeStruct((2, NT, NPT, 16), jnp.float32),
        scratch_types=scratch,
        compiler_params=_SC_PARAMS,
    )
    def edge_kernel(*refs):
        if layer2:
            (dp_h, tf_h, dst_h, src_h, ea_h, t1a_h, t1b_h, t2_h, ab_h,
             y1a_h, y1b_h, out_h,
             agg, dpb, tfb, dstb, eab, t1r, msg, t2b, abb, sem,
             srcb, y1r) = refs
        else:
            (dp_h, tf_h, dst_h, ea_h, t1a_h, t1b_h, t2_h, ab_h, out_h,
             agg, dpb, tfb, dstb, eab, t1r, msg, t2b, abb, sem) = refs

        c = lax.axis_index("c")
        s = lax.axis_index("s")

        pltpu.sync_copy(t2_h.at[c], t2b)
        pltpu.sync_copy(ab_h.at[c], abb)
        _zero_agg(msg, agg, s)
        plsc.subcore_barrier()

        iota16 = lax.iota(jnp.int32, 16)
        nch = (NCH - s + NT - 1) // NT

        def chunk(i, carry):
            base = (s + i * NT) * CH
            cps = [
                pltpu.async_copy(dp_h.at[pl.ds(base, CH)], dpb, sem),
                pltpu.async_copy(tf_h.at[pl.ds(base, CH)], tfb, sem),
                pltpu.async_copy(dst_h.at[pl.ds(base, CH)], dstb.at[0], sem),
                pltpu.async_copy(ea_h.at[pl.ds(base, CH)], eab, sem),
            ]
            if layer2:
                cps.append(pltpu.async_copy(src_h.at[pl.ds(base, CH)], srcb, sem))
            for cp in cps:
                cp.wait()

            @pl.when(c == 0)
            def _():
                pltpu.async_copy(t1a_h.at[dpb], t1r, sem).wait()
                if layer2:
                    pltpu.async_copy(y1a_h.at[srcb], y1r, sem).wait()

            @pl.when(c == 1)
            def _():
                pltpu.async_copy(t1b_h.at[dpb], t1r, sem).wait()
                if layer2:
                    pltpu.async_copy(y1b_h.at[srcb], y1r, sem).wait()

            a0v = abb[0]
            a1v = abb[1]
            a2v = abb[2]
            a3v = abb[3]
            bvv = abb[4]
            for g in range(CH // 16):
                eidx = iota16 + (g * 16)
                tf16 = tfb[pl.ds(g * 16, 16)]
                ea0 = plsc.load_gather(eab, [eidx, _c16(0)])
                ea1 = plsc.load_gather(eab, [eidx, _c16(1)])
                ea2 = plsc.load_gather(eab, [eidx, _c16(2)])
                ea3 = plsc.load_gather(eab, [eidx, _c16(3)])
                for j in range(16):
                    jv = _c16(j)
                    acc = (plsc.load_gather(t1r, [eidx, jv])
                           + plsc.load_gather(t2b, [tf16, jv]))
                    acc = acc + jnp.broadcast_to(a0v[j], (16,)) * ea0
                    acc = acc + jnp.broadcast_to(a1v[j], (16,)) * ea1
                    acc = acc + jnp.broadcast_to(a2v[j], (16,)) * ea2
                    acc = acc + jnp.broadcast_to(a3v[j], (16,)) * ea3
                    acc = acc + jnp.broadcast_to(bvv[j], (16,))
                    if layer2:
                        acc = acc + plsc.load_gather(y1r, [eidx, jv])
                    plsc.store_scatter(msg, [eidx, jv],
                                       jnp.maximum(acc, 0.0))
            pltpu.sync_copy(msg, agg.at[dstb.at[0]], add=True)
            return carry

        lax.fori_loop(0, nch, chunk, 0)
        plsc.subcore_barrier()
        pltpu.sync_copy(agg.at[pl.ds(s * NPT, NPT)], out_h.at[c, s])

    return edge_kernel


_edge_l1 = _make_edge_kernel(False)
_edge_l2 = _make_edge_kernel(True)


@functools.partial(
    pl.kernel,
    mesh=_mesh,
    out_type=(
        jax.ShapeDtypeStruct((2, NT, G, 16), jnp.float32),
        jax.ShapeDtypeStruct((2, NT, G, 16), jnp.float32),
        jax.ShapeDtypeStruct((NT, G, 16), jnp.float32),
    ),
    scratch_types=[
        pltpu.VMEM((CH, 16), jnp.float32),
        pltpu.VMEM((CH, 16), jnp.float32),
        pltpu.VMEM((CH + 16,), jnp.int32),
        pltpu.VMEM((G, 16), jnp.float32),
        pltpu.VMEM((G, 16), jnp.float32),
        pltpu.VMEM((G, 16), jnp.float32),
        pltpu.SemaphoreType.DMA,
    ],
    compiler_params=_SC_PARAMS,
)
def _pool_kernel(p1_h, a2_h, batch_h, sums_o, maxs_o, cnt_o,
                 p1b, a2b, bb, sumt, maxt, cntt, sem):
    c = lax.axis_index("c")
    s = lax.axis_index("s")

    def init(g, carry):
        sumt[g] = jnp.zeros((16,), jnp.float32)
        maxt[g] = jnp.full((16,), -jnp.inf, jnp.float32)
        cntt[g] = jnp.zeros((16,), jnp.float32)
        return carry

    lax.fori_loop(0, G, init, 0)
    one0 = (lax.iota(jnp.int32, 16) == 0).astype(jnp.float32)

    def do_nodes(count, carry):
        def nbody(e, ncarry):
            seg = bb[pl.ds(e, 16)][0]
            row = jnp.maximum(p1b[e] + a2b[e], 0.0)
            sumt[seg] = sumt[seg] + row
            maxt[seg] = jnp.maximum(maxt[seg], row)
            cntt[seg] = cntt[seg] + one0
            return ncarry

        lax.fori_loop(0, count, nbody, 0)
        return carry

    nfull = (N_FULL_NCH - s + NT - 1) // NT

    def chunk(i, carry):
        base = (s + i * NT) * CH
        cps = [
            pltpu.async_copy(p1_h.at[c, pl.ds(base, CH)], p1b, sem),
            pltpu.async_copy(a2_h.at[c, pl.ds(base, CH)], a2b, sem),
            pltpu.async_copy(batch_h.at[pl.ds(base, CH)], bb.at[pl.ds(0, CH)], sem),
        ]
        for cp in cps:
            cp.wait()
        return do_nodes(CH, carry)

    lax.fori_loop(0, nfull, chunk, 0)

    @pl.when(s == TAIL_TILE)
    def _():
        base = N_FULL_NCH * CH
        cps = [
            pltpu.async_copy(p1_h.at[c, pl.ds(base, N_TAIL)],
                             p1b.at[pl.ds(0, N_TAIL)], sem),
            pltpu.async_copy(a2_h.at[c, pl.ds(base, N_TAIL)],
                             a2b.at[pl.ds(0, N_TAIL)], sem),
            pltpu.async_copy(batch_h.at[pl.ds(base, N_TAIL)],
                             bb.at[pl.ds(0, N_TAIL)], sem),
        ]
        for cp in cps:
            cp.wait()
        do_nodes(N_TAIL, 0)

    pltpu.sync_copy(sumt, sums_o.at[c, s])
    pltpu.sync_copy(maxt, maxs_o.at[c, s])

    @pl.when(c == 0)
    def _():
        pltpu.sync_copy(cntt, cnt_o.at[s])


def _tables_body(dpt_ref, tft_ref, we1_ref, we2_ref,
                 t11_ref, t12_ref, t21_ref, t22_ref):
    i = pl.program_id(0)
    tb = dpt_ref[...]
    o1 = jnp.dot(tb, we1_ref[4:20, :], preferred_element_type=jnp.float32, precision=lax.Precision.HIGHEST)
    t11_ref[0] = o1[:, :16]
    t11_ref[1] = o1[:, 16:]
    o2 = jnp.dot(tb, we2_ref[4:20, :], preferred_element_type=jnp.float32, precision=lax.Precision.HIGHEST)
    t12_ref[0] = o2[:, :16]
    t12_ref[1] = o2[:, 16:]

    @pl.when(i == 0)
    def _():
        tt = tft_ref[...]
        u1 = jnp.dot(tt, we1_ref[20:36, :], preferred_element_type=jnp.float32, precision=lax.Precision.HIGHEST)
        t21_ref[0] = u1[:, :16]
        t21_ref[1] = u1[:, 16:]
        u2 = jnp.dot(tt, we2_ref[20:36, :], preferred_element_type=jnp.float32, precision=lax.Precision.HIGHEST)
        t22_ref[0] = u2[:, :16]
        t22_ref[1] = u2[:, 16:]


_TBS = 2048


def _tables_call(dpt, tft, we1, we2):
    grid = (65536 // _TBS,)
    return pl.pallas_call(
        _tables_body,
        grid=grid,
        in_specs=[
            pl.BlockSpec((_TBS, 16), lambda i: (i, 0)),
            pl.BlockSpec((256, 16), lambda i: (0, 0)),
            pl.BlockSpec((36, H), lambda i: (0, 0)),
            pl.BlockSpec((36, H), lambda i: (0, 0)),
        ],
        out_specs=[
            pl.BlockSpec((2, _TBS, 16), lambda i: (0, i, 0)),
            pl.BlockSpec((2, _TBS, 16), lambda i: (0, i, 0)),
            pl.BlockSpec((2, 256, 16), lambda i: (0, 0, 0)),
            pl.BlockSpec((2, 256, 16), lambda i: (0, 0, 0)),
        ],
        out_shape=[
            jax.ShapeDtypeStruct((2, 65536, 16), jnp.float32),
            jax.ShapeDtypeStruct((2, 65536, 16), jnp.float32),
            jax.ShapeDtypeStruct((2, 256, 16), jnp.float32),
            jax.ShapeDtypeStruct((2, 256, 16), jnp.float32),
        ],
    )(dpt, tft, we1, we2)


def _mid_body(x1_ref, ws2_ref, wself2_ref, y1_ref, p1_ref):
    x = jnp.concatenate([x1_ref[0], x1_ref[1]], axis=1)
    y = jnp.dot(x, ws2_ref[...], preferred_element_type=jnp.float32, precision=lax.Precision.HIGHEST)
    p = jnp.dot(x, wself2_ref[...], preferred_element_type=jnp.float32, precision=lax.Precision.HIGHEST)
    y1_ref[0] = y[:, :16]
    y1_ref[1] = y[:, 16:]
    p1_ref[0] = p[:, :16]
    p1_ref[1] = p[:, 16:]


_MBS = 4000


def _mid_call(x1h, ws2, wself2):
    grid = (N // _MBS,)
    return pl.pallas_call(
        _mid_body,
        grid=grid,
        in_specs=[
            pl.BlockSpec((2, _MBS, 16), lambda i: (0, i, 0)),
            pl.BlockSpec((H, H), lambda i: (0, 0)),
            pl.BlockSpec((H, H), lambda i: (0, 0)),
        ],
        out_specs=[
            pl.BlockSpec((2, _MBS, 16), lambda i: (0, i, 0)),
            pl.BlockSpec((2, _MBS, 16), lambda i: (0, i, 0)),
        ],
        out_shape=[
            jax.ShapeDtypeStruct((2, N, 16), jnp.float32),
            jax.ShapeDtypeStruct((2, N, 16), jnp.float32),
        ],
    )(x1h, ws2, wself2)


def _final_body(sums_ref, maxs_ref, cnt_ref, wc_ref, bc_ref, out_ref):
    sm = jnp.sum(sums_ref[...], axis=1)
    sum_full = jnp.concatenate([sm[0], sm[1]], axis=1)
    mx = jnp.max(maxs_ref[...], axis=1)
    max_full = jnp.concatenate([mx[0], mx[1]], axis=1)
    counts = jnp.sum(cnt_ref[...], axis=0)[:, 0:1]
    mean = sum_full / jnp.maximum(counts, 1.0)
    pooled = jnp.concatenate([mean, max_full], axis=1)
    out_ref[...] = (
        jnp.dot(pooled, wc_ref[...], preferred_element_type=jnp.float32, precision=lax.Precision.HIGHEST)
        + bc_ref[...])


def _final_call(sums_p, maxs_p, cnt_p, wc, bc2):
    return pl.pallas_call(
        _final_body,
        out_shape=jax.ShapeDtypeStruct((G, 2), jnp.float32),
    )(sums_p, maxs_p, cnt_p, wc, bc2)


def kernel(edge_attr, dst_ports, tcp_flags, edge_index, batch,
           dst_port_table, tcp_flags_table,
           W_e1, W_s1, W_self1, b1, W_e2, W_s2, W_self2, b2, Wc, bc):
    dp = dst_ports.astype(jnp.int32)
    tf = tcp_flags.astype(jnp.int32)
    src = edge_index[0].astype(jnp.int32)
    dst = edge_index[1].astype(jnp.int32)
    batch = batch.astype(jnp.int32)

    t11, t12, t21, t22 = _tables_call(
        dst_port_table, tcp_flags_table, W_e1, W_e2)

    ab1 = jnp.concatenate([W_e1[:4], b1[None, :]], axis=0)
    ab1h = jnp.stack([ab1[:, :16], ab1[:, 16:]], axis=0)
    ab2 = jnp.concatenate([W_e2[:4], b2[None, :]], axis=0)
    ab2h = jnp.stack([ab2[:, :16], ab2[:, 16:]], axis=0)

    x1h = _edge_l1(dp, tf, dst, edge_attr, t11[0], t11[1], t21, ab1h)
    x1h = x1h.reshape(2, N, 16)

    y1h, p1h = _mid_call(x1h, W_s2, W_self2)

    a2h = _edge_l2(dp, tf, dst, src, edge_attr, t12[0], t12[1], t22, ab2h,
                   y1h[0], y1h[1])
    a2h = a2h.reshape(2, N, 16)

    sums_p, maxs_p, cnt_p = _pool_kernel(p1h, a2h, batch)

    return _final_call(sums_p, maxs_p, cnt_p, Wc, bc.reshape(1, 2))
